# baseline (speedup 1.0000x reference)
import functools

import jax
import jax.numpy as jnp
from jax import lax
from jax.experimental import pallas as pl
from jax.experimental.pallas import tpu as pltpu
from jax.experimental.pallas import tpu_sc as plsc

_Ns = 1000
_bd = 20.0
_bm = 0.1

B = 256
F = 3 * 64 * 64
BB = 128

_info = plsc.get_sparse_core_info()
_NC, _NS = _info.num_cores, _info.num_subcores
_NW = _NC * _NS
_RPW = B // _NW


def _tables():
    ts = jnp.linspace(1e-05, 1.0, _Ns, dtype=jnp.float32)
    betas = (_bm + (_bd - _bm) * ts) / _Ns
    alphas = (1.0 - betas).astype(jnp.float32)
    acp = jnp.cumprod(alphas)
    acp_prev = jnp.concatenate([jnp.ones((1,), jnp.float32), acp[:-1]])
    pv = betas * (1.0 - acp_prev) / (1.0 - acp)
    plv = jnp.log(jnp.clip(pv, 1e-20, None))
    c1 = betas * jnp.sqrt(acp_prev) / (1.0 - acp)
    c2 = (1.0 - acp_prev) * jnp.sqrt(alphas) / (1.0 - acp)
    return (pv.astype(jnp.float32), plv.astype(jnp.float32),
            c1.astype(jnp.float32), c2.astype(jnp.float32))


def _mean_body(i_ref, c1_ref, c2_ref, x_ref, y_ref, o_ref):
    b = pl.program_id(0)
    idx = [i_ref[b * BB + r] for r in range(BB)]
    c1v = jnp.stack([c1_ref[t] for t in idx]).reshape(BB, 1)
    c2v = jnp.stack([c2_ref[t] for t in idx]).reshape(BB, 1)
    o_ref[...] = c1v * x_ref[...] + c2v * y_ref[...]


def _sc_var_body(i_hbm, tab_hbm, out_hbm, idx_v, rows_v, sem):
    wid = lax.axis_index("s") * _NC + lax.axis_index("c")
    base = wid * _RPW
    pltpu.sync_copy(i_hbm.at[pl.ds(base, _RPW)], idx_v)
    pltpu.async_copy(tab_hbm.at[idx_v], rows_v, sem).wait()
    pltpu.sync_copy(rows_v, out_hbm.at[pl.ds(base, _RPW)])


_sc_var = functools.partial(
    pl.kernel,
    mesh=plsc.VectorSubcoreMesh(core_axis_name="c", subcore_axis_name="s"),
    out_type=jax.ShapeDtypeStruct((B, 128), jnp.float32),
    scratch_types=[
        pltpu.VMEM((_RPW,), jnp.int32),
        pltpu.VMEM((_RPW, 128), jnp.float32),
        pltpu.SemaphoreType.DMA,
    ],
)(_sc_var_body)


@jax.jit
def kernel(x0, x_i, i):
    pv, plv, c1, c2 = _tables()
    vtab = jnp.zeros((_Ns, 128), jnp.float32)
    vtab = vtab.at[:, 0].set(pv).at[:, 1].set(plv)

    var_rows = _sc_var(i, vtab)

    smem = pl.BlockSpec(memory_space=pltpu.SMEM)
    mean = pl.pallas_call(
        _mean_body,
        grid=(B // BB,),
        in_specs=[smem, smem, smem,
                  pl.BlockSpec((BB, F), lambda b: (b, 0)),
                  pl.BlockSpec((BB, F), lambda b: (b, 0))],
        out_specs=pl.BlockSpec((BB, F), lambda b: (b, 0)),
        out_shape=jax.ShapeDtypeStruct((B, F), jnp.float32),
    )(i, c1, c2, x0.reshape(B, F), x_i.reshape(B, F))

    posterior_mean = mean.reshape(x0.shape)
    posterior_variance = var_rows[:, 0].reshape(B, 1, 1, 1)
    posterior_log_variance_clipped = var_rows[:, 1].reshape(B, 1, 1, 1)
    return (posterior_mean, posterior_variance,
            posterior_log_variance_clipped)
